# Initial kernel scaffold; baseline (speedup 1.0000x reference)
import functools

import jax
import jax.numpy as jnp
from jax import lax
from jax.experimental import pallas as pl
from jax.experimental.pallas import tpu as pltpu
from jax.experimental.pallas import tpu_sc as plsc

N = 10000
E = 320000
D = 128
EPS = 0.1

NC = 2
NS = 16
NW = NC * NS
C = 80
EPT = E // NW
NCH = EPT // C
RPT = N // NS
DEGW = 16

_mesh = plsc.VectorSubcoreMesh(
    core_axis_name="c", subcore_axis_name="s", num_cores=NC, num_subcores=NS)


@functools.partial(
    pl.kernel,
    out_type=jax.ShapeDtypeStruct((NC * N, DEGW), jnp.float32),
    mesh=_mesh,
    scratch_types=[
        pltpu.VMEM_SHARED((N, DEGW), jnp.float32),
        pltpu.VMEM((NCH, C), jnp.int32),
        pltpu.VMEM((C, DEGW), jnp.float32),
    ],
)
def _deg_kernel(col_hbm, zeros_hbm, ones_hbm, out_hbm, acc, colv, ones_v):
    c = lax.axis_index("c")
    s = lax.axis_index("s")
    wid = c * NS + s
    pltpu.sync_copy(zeros_hbm, acc.at[pl.ds(s * RPT, RPT)])
    pltpu.sync_copy(ones_hbm, ones_v)
    pltpu.sync_copy(col_hbm.at[pl.ds(wid * NCH, NCH)], colv)
    plsc.subcore_barrier()

    def body(j, carry):
        pltpu.sync_copy(ones_v, acc.at[colv.at[j]], add=True)
        return carry

    lax.fori_loop(0, NCH, body, 0)
    plsc.subcore_barrier()
    pltpu.sync_copy(acc.at[pl.ds(s * RPT, RPT)],
                    out_hbm.at[pl.ds(c * N + s * RPT, RPT)])


@functools.partial(
    pl.kernel,
    out_type=jax.ShapeDtypeStruct((NC * N, D), jnp.float32),
    mesh=_mesh,
    scratch_types=[
        pltpu.VMEM_SHARED((N, D), jnp.float32),
        pltpu.VMEM((NCH, C), jnp.int32),
        pltpu.VMEM((NCH, C), jnp.int32),
        pltpu.VMEM((C, D), jnp.float32),
        pltpu.SemaphoreType.DMA,
    ],
)
def _segsum_kernel(hs_hbm, row_hbm, col_hbm, zeros_hbm, out_hbm,
                   acc, rowv, colv, buf, sem):
    c = lax.axis_index("c")
    s = lax.axis_index("s")
    wid = c * NS + s
    pltpu.sync_copy(zeros_hbm, acc.at[pl.ds(s * RPT, RPT)])
    pltpu.sync_copy(row_hbm.at[pl.ds(wid * NCH, NCH)], rowv)
    pltpu.sync_copy(col_hbm.at[pl.ds(wid * NCH, NCH)], colv)
    plsc.subcore_barrier()

    def body(j, carry):
        pltpu.async_copy(hs_hbm.at[rowv.at[j]], buf, sem).wait()
        pltpu.sync_copy(buf, acc.at[colv.at[j]], add=True)
        return carry

    lax.fori_loop(0, NCH, body, 0)
    plsc.subcore_barrier()
    pltpu.sync_copy(acc.at[pl.ds(s * RPT, RPT)],
                    out_hbm.at[pl.ds(c * N + s * RPT, RPT)])


_RB = 1000


def _rows(i):
    return (i, 0)


def _full(i):
    return (0, 0)


def _tc1_body(x_ref, w1_ref, b1_ref, a0_ref, degc_ref, hs_ref, c0_ref, dis_ref):
    h = lax.dot_general(x_ref[...], w1_ref[...], (((1,), (1,)), ((), ())),
                        preferred_element_type=jnp.float32)
    h = jnp.maximum(h + b1_ref[...], 0.0)
    deg = jnp.sum(degc_ref[...], axis=1, keepdims=True) * (1.0 / DEGW) + 1.0
    dis = lax.rsqrt(deg)
    z = lax.dot_general(h, a0_ref[...], (((1,), (1,)), ((), ())),
                        preferred_element_type=jnp.float32)
    g = jax.nn.sigmoid(z)
    c0_ref[...] = (g * (1.0 + EPS) - EPS) * dis
    dis_ref[...] = dis
    hs_ref[...] = h * dis


_tc1 = pl.pallas_call(
    _tc1_body,
    grid=(N // _RB,),
    in_specs=[
        pl.BlockSpec((_RB, D), _rows),
        pl.BlockSpec((D, D), _full),
        pl.BlockSpec((1, D), _full),
        pl.BlockSpec((1, D), _full),
        pl.BlockSpec((_RB, NC * DEGW), _rows),
    ],
    out_specs=[
        pl.BlockSpec((_RB, D), _rows),
        pl.BlockSpec((_RB, 1), _rows),
        pl.BlockSpec((_RB, 1), _rows),
    ],
    out_shape=[
        jax.ShapeDtypeStruct((N, D), jnp.float32),
        jax.ShapeDtypeStruct((N, 1), jnp.float32),
        jax.ShapeDtypeStruct((N, 1), jnp.float32),
    ],
)


def _tc2_body(ta_ref, tb_ref, hs_ref, c0_ref, dis_ref, a1_ref,
              hs1_ref, c1_ref):
    h1 = c0_ref[...] * (ta_ref[...] + tb_ref[...] + hs_ref[...])
    dis = dis_ref[...]
    z = lax.dot_general(h1, a1_ref[...], (((1,), (1,)), ((), ())),
                        preferred_element_type=jnp.float32)
    g = jax.nn.sigmoid(z)
    c1_ref[...] = (g * (1.0 + EPS) - EPS) * dis
    hs1_ref[...] = h1 * dis


_tc2 = pl.pallas_call(
    _tc2_body,
    grid=(N // _RB,),
    in_specs=[
        pl.BlockSpec((_RB, D), _rows),
        pl.BlockSpec((_RB, D), _rows),
        pl.BlockSpec((_RB, D), _rows),
        pl.BlockSpec((_RB, 1), _rows),
        pl.BlockSpec((_RB, 1), _rows),
        pl.BlockSpec((1, D), _full),
    ],
    out_specs=[
        pl.BlockSpec((_RB, D), _rows),
        pl.BlockSpec((_RB, 1), _rows),
    ],
    out_shape=[
        jax.ShapeDtypeStruct((N, D), jnp.float32),
        jax.ShapeDtypeStruct((N, 1), jnp.float32),
    ],
)


def _tc3_body(ta_ref, tb_ref, hs1_ref, c1_ref, w2_ref, b2_ref, out_ref):
    h2 = c1_ref[...] * (ta_ref[...] + tb_ref[...] + hs1_ref[...])
    out_ref[...] = lax.dot_general(h2, w2_ref[...], (((1,), (1,)), ((), ())),
                                   preferred_element_type=jnp.float32) + b2_ref[...]


_tc3 = pl.pallas_call(
    _tc3_body,
    grid=(N // _RB,),
    in_specs=[
        pl.BlockSpec((_RB, D), _rows),
        pl.BlockSpec((_RB, D), _rows),
        pl.BlockSpec((_RB, D), _rows),
        pl.BlockSpec((_RB, 1), _rows),
        pl.BlockSpec((D, D), _full),
        pl.BlockSpec((1, D), _full),
    ],
    out_specs=pl.BlockSpec((_RB, D), _rows),
    out_shape=jax.ShapeDtypeStruct((N, D), jnp.float32),
)


def kernel(x, edge_index, W1, b1, att0, att1, W2, b2):
    row2 = edge_index[0].reshape(E // C, C)
    col2 = edge_index[1].reshape(E // C, C)
    zeros_d = jnp.zeros((RPT, DEGW), jnp.float32)
    ones_d = jnp.ones((C, DEGW), jnp.float32)
    zeros_f = jnp.zeros((RPT, D), jnp.float32)
    b1r = b1.reshape(1, D)
    b2r = b2.reshape(1, D)

    degp = _deg_kernel(col2, zeros_d, ones_d)
    degc = degp.reshape(NC, N, DEGW).transpose(1, 0, 2).reshape(N, NC * DEGW)

    hs, c0, dis = _tc1(x, W1, b1r, att0, degc)
    t1 = _segsum_kernel(hs, row2, col2, zeros_f)
    hs1, c1 = _tc2(t1[:N], t1[N:], hs, c0, dis, att1)
    t2 = _segsum_kernel(hs1, row2, col2, zeros_f)
    out = _tc3(t2[:N], t2[N:], hs1, c1, W2, b2r)
    return out


# speedup vs baseline: 21.3782x; 21.3782x over previous
import functools

import jax
import jax.numpy as jnp
from jax import lax
from jax.experimental import pallas as pl
from jax.experimental.pallas import tpu as pltpu
from jax.experimental.pallas import tpu_sc as plsc

N = 10000
E = 320000
D = 128
EPS = 0.1

NC = 2
NS = 16
NW = NC * NS
C = 80
EPT = E // NW
NCH = EPT // C
SLAB = 624
TAIL = N - NS * SLAB
DEGW = 128

_mesh = plsc.VectorSubcoreMesh(
    core_axis_name="c", subcore_axis_name="s", num_cores=NC, num_subcores=NS)


@functools.partial(
    pl.kernel,
    out_type=jax.ShapeDtypeStruct((NC * N, DEGW), jnp.float32),
    mesh=_mesh,
    scratch_types=[
        pltpu.VMEM_SHARED((N, DEGW), jnp.float32),
        pltpu.VMEM((NCH, C), jnp.int32),
        pltpu.VMEM((C, DEGW), jnp.float32),
    ],
)
def _deg_kernel(col_hbm, zeros_hbm, ones_hbm, out_hbm, acc, colv, ones_v):
    c = lax.axis_index("c")
    s = lax.axis_index("s")
    wid = c * NS + s
    pltpu.sync_copy(zeros_hbm.at[pl.ds(0, SLAB)], acc.at[pl.ds(s * SLAB, SLAB)])

    @pl.when(s == NS - 1)
    def _():
        pltpu.sync_copy(zeros_hbm.at[pl.ds(0, TAIL)],
                        acc.at[pl.ds(NS * SLAB, TAIL)])

    pltpu.sync_copy(ones_hbm, ones_v)
    pltpu.sync_copy(col_hbm.at[wid], colv)
    plsc.subcore_barrier()

    def body(j, carry):
        pltpu.sync_copy(ones_v, acc.at[colv.at[j]], add=True)
        return carry

    lax.fori_loop(0, NCH, body, 0)
    plsc.subcore_barrier()
    pltpu.sync_copy(acc.at[pl.ds(s * SLAB, SLAB)],
                    out_hbm.at[pl.ds(c * N + s * SLAB, SLAB)])

    @pl.when(s == NS - 1)
    def _():
        pltpu.sync_copy(acc.at[pl.ds(NS * SLAB, TAIL)],
                        out_hbm.at[pl.ds(c * N + NS * SLAB, TAIL)])


@functools.partial(
    pl.kernel,
    out_type=jax.ShapeDtypeStruct((NC * N, D), jnp.float32),
    mesh=_mesh,
    scratch_types=[
        pltpu.VMEM_SHARED((N, D), jnp.float32),
        pltpu.VMEM((NCH, C), jnp.int32),
        pltpu.VMEM((NCH, C), jnp.int32),
        pltpu.VMEM((C, D), jnp.float32),
        pltpu.SemaphoreType.DMA,
    ],
)
def _segsum_kernel(hs_hbm, row_hbm, col_hbm, zeros_hbm, out_hbm,
                   acc, rowv, colv, buf, sem):
    c = lax.axis_index("c")
    s = lax.axis_index("s")
    wid = c * NS + s
    pltpu.sync_copy(zeros_hbm.at[pl.ds(0, SLAB)], acc.at[pl.ds(s * SLAB, SLAB)])

    @pl.when(s == NS - 1)
    def _():
        pltpu.sync_copy(zeros_hbm.at[pl.ds(0, TAIL)],
                        acc.at[pl.ds(NS * SLAB, TAIL)])

    pltpu.sync_copy(row_hbm.at[wid], rowv)
    pltpu.sync_copy(col_hbm.at[wid], colv)
    plsc.subcore_barrier()

    def body(j, carry):
        pltpu.async_copy(hs_hbm.at[rowv.at[j]], buf, sem).wait()
        pltpu.sync_copy(buf, acc.at[colv.at[j]], add=True)
        return carry

    lax.fori_loop(0, NCH, body, 0)
    plsc.subcore_barrier()
    pltpu.sync_copy(acc.at[pl.ds(s * SLAB, SLAB)],
                    out_hbm.at[pl.ds(c * N + s * SLAB, SLAB)])

    @pl.when(s == NS - 1)
    def _():
        pltpu.sync_copy(acc.at[pl.ds(NS * SLAB, TAIL)],
                        out_hbm.at[pl.ds(c * N + NS * SLAB, TAIL)])


_RB = 1000


def _rows(i):
    return (i, 0)


def _full(i):
    return (0, 0)


def _tc1_body(x_ref, w1_ref, b1_ref, a0_ref, degc_ref, hs_ref, c0_ref, dis_ref):
    h = lax.dot_general(x_ref[...], w1_ref[...], (((1,), (1,)), ((), ())),
                        preferred_element_type=jnp.float32)
    h = jnp.maximum(h + b1_ref[...], 0.0)
    deg = jnp.sum(degc_ref[...], axis=1, keepdims=True) * (1.0 / DEGW) + 1.0
    dis = lax.rsqrt(deg)
    z = lax.dot_general(h, a0_ref[...], (((1,), (1,)), ((), ())),
                        preferred_element_type=jnp.float32)
    g = jax.nn.sigmoid(z)
    c0_ref[...] = (g * (1.0 + EPS) - EPS) * dis
    dis_ref[...] = dis
    hs_ref[...] = h * dis


_tc1 = pl.pallas_call(
    _tc1_body,
    grid=(N // _RB,),
    in_specs=[
        pl.BlockSpec((_RB, D), _rows),
        pl.BlockSpec((D, D), _full),
        pl.BlockSpec((1, D), _full),
        pl.BlockSpec((1, D), _full),
        pl.BlockSpec((_RB, NC * DEGW), _rows),
    ],
    out_specs=[
        pl.BlockSpec((_RB, D), _rows),
        pl.BlockSpec((_RB, 1), _rows),
        pl.BlockSpec((_RB, 1), _rows),
    ],
    out_shape=[
        jax.ShapeDtypeStruct((N, D), jnp.float32),
        jax.ShapeDtypeStruct((N, 1), jnp.float32),
        jax.ShapeDtypeStruct((N, 1), jnp.float32),
    ],
)


def _tc2_body(ta_ref, tb_ref, hs_ref, c0_ref, dis_ref, a1_ref,
              hs1_ref, c1_ref):
    h1 = c0_ref[...] * (ta_ref[...] + tb_ref[...] + hs_ref[...])
    dis = dis_ref[...]
    z = lax.dot_general(h1, a1_ref[...], (((1,), (1,)), ((), ())),
                        preferred_element_type=jnp.float32)
    g = jax.nn.sigmoid(z)
    c1_ref[...] = (g * (1.0 + EPS) - EPS) * dis
    hs1_ref[...] = h1 * dis


_tc2 = pl.pallas_call(
    _tc2_body,
    grid=(N // _RB,),
    in_specs=[
        pl.BlockSpec((_RB, D), _rows),
        pl.BlockSpec((_RB, D), _rows),
        pl.BlockSpec((_RB, D), _rows),
        pl.BlockSpec((_RB, 1), _rows),
        pl.BlockSpec((_RB, 1), _rows),
        pl.BlockSpec((1, D), _full),
    ],
    out_specs=[
        pl.BlockSpec((_RB, D), _rows),
        pl.BlockSpec((_RB, 1), _rows),
    ],
    out_shape=[
        jax.ShapeDtypeStruct((N, D), jnp.float32),
        jax.ShapeDtypeStruct((N, 1), jnp.float32),
    ],
)


def _tc3_body(ta_ref, tb_ref, hs1_ref, c1_ref, w2_ref, b2_ref, out_ref):
    h2 = c1_ref[...] * (ta_ref[...] + tb_ref[...] + hs1_ref[...])
    out_ref[...] = lax.dot_general(h2, w2_ref[...], (((1,), (1,)), ((), ())),
                                   preferred_element_type=jnp.float32) + b2_ref[...]


_tc3 = pl.pallas_call(
    _tc3_body,
    grid=(N // _RB,),
    in_specs=[
        pl.BlockSpec((_RB, D), _rows),
        pl.BlockSpec((_RB, D), _rows),
        pl.BlockSpec((_RB, D), _rows),
        pl.BlockSpec((_RB, 1), _rows),
        pl.BlockSpec((D, D), _full),
        pl.BlockSpec((1, D), _full),
    ],
    out_specs=pl.BlockSpec((_RB, D), _rows),
    out_shape=jax.ShapeDtypeStruct((N, D), jnp.float32),
)


def kernel(x, edge_index, W1, b1, att0, att1, W2, b2):
    row2 = edge_index[0].reshape(NW, NCH, C)
    col2 = edge_index[1].reshape(NW, NCH, C)
    zeros_d = jnp.zeros((SLAB, DEGW), jnp.float32)
    ones_d = jnp.ones((C, DEGW), jnp.float32)
    zeros_f = jnp.zeros((SLAB, D), jnp.float32)
    b1r = b1.reshape(1, D)
    b2r = b2.reshape(1, D)

    degp = _deg_kernel(col2, zeros_d, ones_d)
    degc = degp.reshape(NC, N, DEGW).transpose(1, 0, 2).reshape(N, NC * DEGW)

    hs, c0, dis = _tc1(x, W1, b1r, att0, degc)
    t1 = _segsum_kernel(hs, row2, col2, zeros_f)
    hs1, c1 = _tc2(t1[:N], t1[N:], hs, c0, dis, att1)
    t2 = _segsum_kernel(hs1, row2, col2, zeros_f)
    out = _tc3(t2[:N], t2[N:], hs1, c1, W2, b2r)
    return out


# speedup vs baseline: 32.6711x; 1.5282x over previous
import functools

import jax
import jax.numpy as jnp
from jax import lax
from jax.experimental import pallas as pl
from jax.experimental.pallas import tpu as pltpu
from jax.experimental.pallas import tpu_sc as plsc

N = 10000
E = 320000
D = 128
EPS = 0.1

NC = 2
NS = 16
NW = NC * NS
C = 80
EPT = E // NW
NCH = EPT // C
SLAB = 624
TAIL = N - NS * SLAB
DEGW = 128

_mesh = plsc.VectorSubcoreMesh(
    core_axis_name="c", subcore_axis_name="s", num_cores=NC, num_subcores=NS)


@functools.partial(
    pl.kernel,
    out_type=jax.ShapeDtypeStruct((NC * N, DEGW), jnp.float32),
    mesh=_mesh,
    scratch_types=[
        pltpu.VMEM_SHARED((N, DEGW), jnp.float32),
        pltpu.VMEM((NCH, C), jnp.int32),
        pltpu.VMEM((C, DEGW), jnp.float32),
    ],
)
def _deg_kernel(col_hbm, zeros_hbm, ones_hbm, out_hbm, acc, colv, ones_v):
    c = lax.axis_index("c")
    s = lax.axis_index("s")
    wid = c * NS + s
    pltpu.sync_copy(zeros_hbm.at[pl.ds(0, SLAB)], acc.at[pl.ds(s * SLAB, SLAB)])

    @pl.when(s == NS - 1)
    def _():
        pltpu.sync_copy(zeros_hbm.at[pl.ds(0, TAIL)],
                        acc.at[pl.ds(NS * SLAB, TAIL)])

    pltpu.sync_copy(ones_hbm, ones_v)
    pltpu.sync_copy(col_hbm.at[wid], colv)
    plsc.subcore_barrier()

    def body(j, carry):
        pltpu.sync_copy(ones_v, acc.at[colv.at[j]], add=True)
        return carry

    lax.fori_loop(0, NCH, body, 0)
    plsc.subcore_barrier()
    pltpu.sync_copy(acc.at[pl.ds(s * SLAB, SLAB)],
                    out_hbm.at[pl.ds(c * N + s * SLAB, SLAB)])

    @pl.when(s == NS - 1)
    def _():
        pltpu.sync_copy(acc.at[pl.ds(NS * SLAB, TAIL)],
                        out_hbm.at[pl.ds(c * N + NS * SLAB, TAIL)])


NBUF = 2
PH0 = 64
PH1 = NCH - PH0


@functools.partial(
    pl.kernel,
    out_type=jax.ShapeDtypeStruct((NC * N, D), jnp.float32),
    mesh=_mesh,
    scratch_types=[
        pltpu.VMEM_SHARED((N, D), jnp.float32),
        pltpu.VMEM((PH0, C), jnp.int32),
        pltpu.VMEM((PH0, C), jnp.int32),
        pltpu.VMEM((NBUF, C, D), jnp.float32),
    ] + [pltpu.SemaphoreType.DMA] * NBUF,
)
def _segsum_kernel(hs_hbm, row_hbm, col_hbm, zeros_hbm, out_hbm,
                   acc, rowv, colv, buf, sem0, sem1):
    sems = (sem0, sem1)
    c = lax.axis_index("c")
    s = lax.axis_index("s")
    wid = c * NS + s
    pltpu.sync_copy(zeros_hbm.at[pl.ds(0, SLAB)], acc.at[pl.ds(s * SLAB, SLAB)])

    @pl.when(s == NS - 1)
    def _():
        pltpu.sync_copy(zeros_hbm.at[pl.ds(0, TAIL)],
                        acc.at[pl.ds(NS * SLAB, TAIL)])

    plsc.subcore_barrier()

    for base, count in ((0, PH0), (PH0, PH1)):
        pltpu.sync_copy(row_hbm.at[wid].at[pl.ds(base, count)],
                        rowv.at[pl.ds(0, count)])
        pltpu.sync_copy(col_hbm.at[wid].at[pl.ds(base, count)],
                        colv.at[pl.ds(0, count)])
        for b in range(NBUF):
            pltpu.async_copy(hs_hbm.at[rowv.at[b]], buf.at[b], sems[b])

        grp = count // NBUF

        def group(g, carry):
            for b in range(NBUF):
                j = g * NBUF + b
                pltpu.make_async_copy(hs_hbm.at[rowv.at[j]], buf.at[b],
                                      sems[b]).wait()
                pltpu.sync_copy(buf.at[b], acc.at[colv.at[j]], add=True)

                @pl.when(j + NBUF < count)
                def _():
                    pltpu.async_copy(hs_hbm.at[rowv.at[j + NBUF]], buf.at[b],
                                     sems[b])
            return carry

        lax.fori_loop(0, grp, group, 0)
        for j in range(grp * NBUF, count):
            b = j % NBUF
            pltpu.make_async_copy(hs_hbm.at[rowv.at[j]], buf.at[b],
                                  sems[b]).wait()
            pltpu.sync_copy(buf.at[b], acc.at[colv.at[j]], add=True)
    plsc.subcore_barrier()
    pltpu.sync_copy(acc.at[pl.ds(s * SLAB, SLAB)],
                    out_hbm.at[pl.ds(c * N + s * SLAB, SLAB)])

    @pl.when(s == NS - 1)
    def _():
        pltpu.sync_copy(acc.at[pl.ds(NS * SLAB, TAIL)],
                        out_hbm.at[pl.ds(c * N + NS * SLAB, TAIL)])


_RB = 1000


def _rows(i):
    return (i, 0)


def _rows2(i):
    return (i + N // 1000, 0)


def _full(i):
    return (0, 0)


def _tc1_body(x_ref, w1_ref, b1_ref, a0_ref, dega_ref, degb_ref,
              hs_ref, c0_ref, dis_ref):
    h = lax.dot_general(x_ref[...], w1_ref[...], (((1,), (1,)), ((), ())),
                        preferred_element_type=jnp.float32)
    h = jnp.maximum(h + b1_ref[...], 0.0)
    deg = (jnp.sum(dega_ref[...], axis=1, keepdims=True) +
           jnp.sum(degb_ref[...], axis=1, keepdims=True)) * (1.0 / DEGW) + 1.0
    dis = lax.rsqrt(deg)
    z = lax.dot_general(h, a0_ref[...], (((1,), (1,)), ((), ())),
                        preferred_element_type=jnp.float32)
    g = jax.nn.sigmoid(z)
    c0_ref[...] = (g * (1.0 + EPS) - EPS) * dis
    dis_ref[...] = dis
    hs_ref[...] = h * dis


_tc1 = pl.pallas_call(
    _tc1_body,
    grid=(N // _RB,),
    in_specs=[
        pl.BlockSpec((_RB, D), _rows),
        pl.BlockSpec((D, D), _full),
        pl.BlockSpec((1, D), _full),
        pl.BlockSpec((1, D), _full),
        pl.BlockSpec((_RB, DEGW), _rows),
        pl.BlockSpec((_RB, DEGW), _rows2),
    ],
    out_specs=[
        pl.BlockSpec((_RB, D), _rows),
        pl.BlockSpec((_RB, 1), _rows),
        pl.BlockSpec((_RB, 1), _rows),
    ],
    out_shape=[
        jax.ShapeDtypeStruct((N, D), jnp.float32),
        jax.ShapeDtypeStruct((N, 1), jnp.float32),
        jax.ShapeDtypeStruct((N, 1), jnp.float32),
    ],
)


def _tc2_body(ta_ref, tb_ref, hs_ref, c0_ref, dis_ref, a1_ref,
              hs1_ref, c1_ref):
    h1 = c0_ref[...] * (ta_ref[...] + tb_ref[...] + hs_ref[...])
    dis = dis_ref[...]
    z = lax.dot_general(h1, a1_ref[...], (((1,), (1,)), ((), ())),
                        preferred_element_type=jnp.float32)
    g = jax.nn.sigmoid(z)
    c1_ref[...] = (g * (1.0 + EPS) - EPS) * dis
    hs1_ref[...] = h1 * dis


_tc2 = pl.pallas_call(
    _tc2_body,
    grid=(N // _RB,),
    in_specs=[
        pl.BlockSpec((_RB, D), _rows),
        pl.BlockSpec((_RB, D), _rows2),
        pl.BlockSpec((_RB, D), _rows),
        pl.BlockSpec((_RB, 1), _rows),
        pl.BlockSpec((_RB, 1), _rows),
        pl.BlockSpec((1, D), _full),
    ],
    out_specs=[
        pl.BlockSpec((_RB, D), _rows),
        pl.BlockSpec((_RB, 1), _rows),
    ],
    out_shape=[
        jax.ShapeDtypeStruct((N, D), jnp.float32),
        jax.ShapeDtypeStruct((N, 1), jnp.float32),
    ],
)


def _tc3_body(ta_ref, tb_ref, hs1_ref, c1_ref, w2_ref, b2_ref, out_ref):
    h2 = c1_ref[...] * (ta_ref[...] + tb_ref[...] + hs1_ref[...])
    out_ref[...] = lax.dot_general(h2, w2_ref[...], (((1,), (1,)), ((), ())),
                                   preferred_element_type=jnp.float32) + b2_ref[...]


_tc3 = pl.pallas_call(
    _tc3_body,
    grid=(N // _RB,),
    in_specs=[
        pl.BlockSpec((_RB, D), _rows),
        pl.BlockSpec((_RB, D), _rows2),
        pl.BlockSpec((_RB, D), _rows),
        pl.BlockSpec((_RB, 1), _rows),
        pl.BlockSpec((D, D), _full),
        pl.BlockSpec((1, D), _full),
    ],
    out_specs=pl.BlockSpec((_RB, D), _rows),
    out_shape=jax.ShapeDtypeStruct((N, D), jnp.float32),
)


def kernel(x, edge_index, W1, b1, att0, att1, W2, b2):
    row2 = edge_index[0].reshape(NW, NCH, C)
    col2 = edge_index[1].reshape(NW, NCH, C)
    zeros_d = jnp.zeros((SLAB, DEGW), jnp.float32)
    ones_d = jnp.ones((C, DEGW), jnp.float32)
    zeros_f = jnp.zeros((SLAB, D), jnp.float32)
    b1r = b1.reshape(1, D)
    b2r = b2.reshape(1, D)

    degp = _deg_kernel(col2, zeros_d, ones_d)
    hs, c0, dis = _tc1(x, W1, b1r, att0, degp, degp)
    t1 = _segsum_kernel(hs, row2, col2, zeros_f)
    hs1, c1 = _tc2(t1, t1, hs, c0, dis, att1)
    t2 = _segsum_kernel(hs1, row2, col2, zeros_f)
    out = _tc3(t2, t2, hs1, c1, W2, b2r)
    return out


# speedup vs baseline: 35.6540x; 1.0913x over previous
import functools

import jax
import jax.numpy as jnp
from jax import lax
from jax.experimental import pallas as pl
from jax.experimental.pallas import tpu as pltpu
from jax.experimental.pallas import tpu_sc as plsc

N = 10000
E = 320000
D = 128
EPS = 0.1

NC = 2
NS = 16
NW = NC * NS
C = 80
EPT = E // NW
NCH = EPT // C
SLAB = 624
TAIL = N - NS * SLAB
DEGW = 128

_mesh = plsc.VectorSubcoreMesh(
    core_axis_name="c", subcore_axis_name="s", num_cores=NC, num_subcores=NS)


@functools.partial(
    pl.kernel,
    out_type=jax.ShapeDtypeStruct((NC * N, DEGW), jnp.float32),
    mesh=_mesh,
    scratch_types=[
        pltpu.VMEM_SHARED((N, DEGW), jnp.float32),
        pltpu.VMEM((NCH, C), jnp.int32),
        pltpu.VMEM((C, DEGW), jnp.float32),
        pltpu.SemaphoreType.DMA,
        pltpu.SemaphoreType.DMA,
    ],
)
def _deg_kernel(col_hbm, zeros_hbm, ones_hbm, out_hbm, acc, colv, ones_v,
                dsem0, dsem1):
    dsems = (dsem0, dsem1)
    c = lax.axis_index("c")
    s = lax.axis_index("s")
    wid = c * NS + s
    pltpu.sync_copy(zeros_hbm.at[pl.ds(0, SLAB)], acc.at[pl.ds(s * SLAB, SLAB)])

    @pl.when(s == NS - 1)
    def _():
        pltpu.sync_copy(zeros_hbm.at[pl.ds(0, TAIL)],
                        acc.at[pl.ds(NS * SLAB, TAIL)])

    pltpu.sync_copy(ones_hbm, ones_v)
    pltpu.sync_copy(col_hbm.at[wid], colv)
    plsc.subcore_barrier()

    for b in range(2):
        pltpu.async_copy(ones_v, acc.at[colv.at[b]], dsems[b], add=True)

    def body(g, carry):
        for b in range(2):
            j = g * 2 + b
            pltpu.make_async_copy(ones_v, acc.at[colv.at[j]], dsems[b]).wait()

            @pl.when(j + 2 < NCH)
            def _():
                pltpu.async_copy(ones_v, acc.at[colv.at[j + 2]], dsems[b],
                                 add=True)
        return carry

    lax.fori_loop(0, NCH // 2, body, 0)
    for j in range((NCH // 2) * 2, NCH):
        pltpu.make_async_copy(ones_v, acc.at[colv.at[j]], dsems[j % 2]).wait()
    plsc.subcore_barrier()
    pltpu.sync_copy(acc.at[pl.ds(s * SLAB, SLAB)],
                    out_hbm.at[pl.ds(c * N + s * SLAB, SLAB)])

    @pl.when(s == NS - 1)
    def _():
        pltpu.sync_copy(acc.at[pl.ds(NS * SLAB, TAIL)],
                        out_hbm.at[pl.ds(c * N + NS * SLAB, TAIL)])


NBUF = 3
PH = 32
_PHASES = tuple((b, min(PH, NCH - b)) for b in range(0, NCH, PH))


@functools.partial(
    pl.kernel,
    out_type=jax.ShapeDtypeStruct((NC * N, D), jnp.float32),
    mesh=_mesh,
    scratch_types=[
        pltpu.VMEM_SHARED((N, D), jnp.float32),
        pltpu.VMEM((PH, C), jnp.int32),
        pltpu.VMEM((PH, C), jnp.int32),
        pltpu.VMEM((NBUF, C, D), jnp.float32),
    ] + [pltpu.SemaphoreType.DMA] * NBUF,
)
def _segsum_kernel(hs_hbm, row_hbm, col_hbm, zeros_hbm, out_hbm,
                   acc, rowv, colv, buf, sem0, sem1, sem2):
    sems = (sem0, sem1, sem2)
    c = lax.axis_index("c")
    s = lax.axis_index("s")
    wid = c * NS + s
    pltpu.sync_copy(zeros_hbm.at[pl.ds(0, SLAB)], acc.at[pl.ds(s * SLAB, SLAB)])

    @pl.when(s == NS - 1)
    def _():
        pltpu.sync_copy(zeros_hbm.at[pl.ds(0, TAIL)],
                        acc.at[pl.ds(NS * SLAB, TAIL)])

    plsc.subcore_barrier()

    for base, count in _PHASES:
        pltpu.sync_copy(row_hbm.at[wid].at[pl.ds(base, count)],
                        rowv.at[pl.ds(0, count)])
        pltpu.sync_copy(col_hbm.at[wid].at[pl.ds(base, count)],
                        colv.at[pl.ds(0, count)])
        for b in range(NBUF):
            pltpu.async_copy(hs_hbm.at[rowv.at[b]], buf.at[b], sems[b])

        grp = count // NBUF

        def group(g, carry):
            for b in range(NBUF):
                j = g * NBUF + b
                pltpu.make_async_copy(hs_hbm.at[rowv.at[j]], buf.at[b],
                                      sems[b]).wait()
                pltpu.sync_copy(buf.at[b], acc.at[colv.at[j]], add=True)

                @pl.when(j + NBUF < count)
                def _():
                    pltpu.async_copy(hs_hbm.at[rowv.at[j + NBUF]], buf.at[b],
                                     sems[b])
            return carry

        lax.fori_loop(0, grp, group, 0)
        for j in range(grp * NBUF, count):
            b = j % NBUF
            pltpu.make_async_copy(hs_hbm.at[rowv.at[j]], buf.at[b],
                                  sems[b]).wait()
            pltpu.sync_copy(buf.at[b], acc.at[colv.at[j]], add=True)
    plsc.subcore_barrier()
    pltpu.sync_copy(acc.at[pl.ds(s * SLAB, SLAB)],
                    out_hbm.at[pl.ds(c * N + s * SLAB, SLAB)])

    @pl.when(s == NS - 1)
    def _():
        pltpu.sync_copy(acc.at[pl.ds(NS * SLAB, TAIL)],
                        out_hbm.at[pl.ds(c * N + NS * SLAB, TAIL)])


_RB = 1000


def _rows(i):
    return (i, 0)


def _rows2(i):
    return (i + N // 1000, 0)


def _full(i):
    return (0, 0)


def _tc1_body(x_ref, w1_ref, b1_ref, a0_ref, dega_ref, degb_ref,
              hs_ref, c0_ref, dis_ref):
    h = lax.dot_general(x_ref[...], w1_ref[...], (((1,), (1,)), ((), ())),
                        preferred_element_type=jnp.float32)
    h = jnp.maximum(h + b1_ref[...], 0.0)
    deg = (jnp.sum(dega_ref[...], axis=1, keepdims=True) +
           jnp.sum(degb_ref[...], axis=1, keepdims=True)) * (1.0 / DEGW) + 1.0
    dis = lax.rsqrt(deg)
    z = lax.dot_general(h, a0_ref[...], (((1,), (1,)), ((), ())),
                        preferred_element_type=jnp.float32)
    g = jax.nn.sigmoid(z)
    c0_ref[...] = (g * (1.0 + EPS) - EPS) * dis
    dis_ref[...] = dis
    hs_ref[...] = h * dis


_tc1 = pl.pallas_call(
    _tc1_body,
    grid=(N // _RB,),
    in_specs=[
        pl.BlockSpec((_RB, D), _rows),
        pl.BlockSpec((D, D), _full),
        pl.BlockSpec((1, D), _full),
        pl.BlockSpec((1, D), _full),
        pl.BlockSpec((_RB, DEGW), _rows),
        pl.BlockSpec((_RB, DEGW), _rows2),
    ],
    out_specs=[
        pl.BlockSpec((_RB, D), _rows),
        pl.BlockSpec((_RB, 1), _rows),
        pl.BlockSpec((_RB, 1), _rows),
    ],
    out_shape=[
        jax.ShapeDtypeStruct((N, D), jnp.float32),
        jax.ShapeDtypeStruct((N, 1), jnp.float32),
        jax.ShapeDtypeStruct((N, 1), jnp.float32),
    ],
)


def _tc2_body(ta_ref, tb_ref, hs_ref, c0_ref, dis_ref, a1_ref,
              hs1_ref, c1_ref):
    h1 = c0_ref[...] * (ta_ref[...] + tb_ref[...] + hs_ref[...])
    dis = dis_ref[...]
    z = lax.dot_general(h1, a1_ref[...], (((1,), (1,)), ((), ())),
                        preferred_element_type=jnp.float32)
    g = jax.nn.sigmoid(z)
    c1_ref[...] = (g * (1.0 + EPS) - EPS) * dis
    hs1_ref[...] = h1 * dis


_tc2 = pl.pallas_call(
    _tc2_body,
    grid=(N // _RB,),
    in_specs=[
        pl.BlockSpec((_RB, D), _rows),
        pl.BlockSpec((_RB, D), _rows2),
        pl.BlockSpec((_RB, D), _rows),
        pl.BlockSpec((_RB, 1), _rows),
        pl.BlockSpec((_RB, 1), _rows),
        pl.BlockSpec((1, D), _full),
    ],
    out_specs=[
        pl.BlockSpec((_RB, D), _rows),
        pl.BlockSpec((_RB, 1), _rows),
    ],
    out_shape=[
        jax.ShapeDtypeStruct((N, D), jnp.float32),
        jax.ShapeDtypeStruct((N, 1), jnp.float32),
    ],
)


def _tc3_body(ta_ref, tb_ref, hs1_ref, c1_ref, w2_ref, b2_ref, out_ref):
    h2 = c1_ref[...] * (ta_ref[...] + tb_ref[...] + hs1_ref[...])
    out_ref[...] = lax.dot_general(h2, w2_ref[...], (((1,), (1,)), ((), ())),
                                   preferred_element_type=jnp.float32) + b2_ref[...]


_tc3 = pl.pallas_call(
    _tc3_body,
    grid=(N // _RB,),
    in_specs=[
        pl.BlockSpec((_RB, D), _rows),
        pl.BlockSpec((_RB, D), _rows2),
        pl.BlockSpec((_RB, D), _rows),
        pl.BlockSpec((_RB, 1), _rows),
        pl.BlockSpec((D, D), _full),
        pl.BlockSpec((1, D), _full),
    ],
    out_specs=pl.BlockSpec((_RB, D), _rows),
    out_shape=jax.ShapeDtypeStruct((N, D), jnp.float32),
)


def kernel(x, edge_index, W1, b1, att0, att1, W2, b2):
    row2 = edge_index[0].reshape(NW, NCH, C)
    col2 = edge_index[1].reshape(NW, NCH, C)
    zeros_d = jnp.zeros((SLAB, DEGW), jnp.float32)
    ones_d = jnp.ones((C, DEGW), jnp.float32)
    zeros_f = jnp.zeros((SLAB, D), jnp.float32)
    b1r = b1.reshape(1, D)
    b2r = b2.reshape(1, D)

    degp = _deg_kernel(col2, zeros_d, ones_d)
    hs, c0, dis = _tc1(x, W1, b1r, att0, degp, degp)
    t1 = _segsum_kernel(hs, row2, col2, zeros_f)
    hs1, c1 = _tc2(t1, t1, hs, c0, dis, att1)
    t2 = _segsum_kernel(hs1, row2, col2, zeros_f)
    out = _tc3(t2, t2, hs1, c1, W2, b2r)
    return out
